# baseline (speedup 1.0000x reference)
import functools

import jax
import jax.numpy as jnp
from jax import lax
from jax.experimental import pallas as pl
from jax.experimental.pallas import tpu as pltpu
from jax.experimental.pallas import tpu_sc as plsc

B, L, D = 4096, 200, 64
V = 1000000
DP = 128
NC, NS = 2, 16
NW = NC * NS
BT = B // NW
LANES = 16
NBUF = 3
NBUF_B = 4

_mesh = plsc.VectorSubcoreMesh(core_axis_name="c", subcore_axis_name="s")


@functools.partial(
    pl.kernel,
    out_type=jax.ShapeDtypeStruct((L, D, B), jnp.float32),
    mesh=_mesh,
    compiler_params=pltpu.CompilerParams(
        use_tc_tiling_on_sc=True, needs_layout_passes=False
    ),
    scratch_types=[
        pltpu.VMEM((L, BT), jnp.int32),
        pltpu.VMEM((NBUF_B * BT, DP), jnp.float32),
        pltpu.VMEM((NBUF_B * D, BT), jnp.float32),
        [pltpu.SemaphoreType.DMA] * NBUF_B,
        [pltpu.SemaphoreType.DMA] * NBUF_B,
    ],
)
def _gather(idx_hbm, table_hbm, out_hbm, idx_v, rows_v, tile_v, gsems, ssems):
  wid = lax.axis_index("s") * NC + lax.axis_index("c")
  b0 = wid * BT
  pltpu.sync_copy(idx_hbm.at[:, pl.ds(b0, BT)], idx_v)

  def start_gather(l, b):
    pltpu.async_copy(
        table_hbm.at[idx_v.at[l]], rows_v.at[pl.ds(b * BT, BT)], gsems[b]
    )

  def wait_gather(b):
    pltpu.make_async_copy(
        table_hbm.at[pl.ds(0, BT)], rows_v.at[pl.ds(0, BT)], gsems[b]
    ).wait()

  def wait_store(b):
    pltpu.make_async_copy(
        tile_v.at[pl.ds(0, D)], out_hbm.at[0, :, pl.ds(0, BT)], ssems[b]
    ).wait()

  def dispatch(l, fn):
    for b in range(NBUF_B):
      @pl.when(lax.rem(l, NBUF_B) == b)
      def _(b=b):
        fn(b)

  base16 = lax.iota(jnp.int32, LANES)

  start_gather(0, 0)
  start_gather(1, 1)
  start_gather(2, 2)

  def body(l, carry):
    b = lax.rem(l, NBUF_B)
    dispatch(l, wait_gather)

    @pl.when(l >= NBUF_B)
    def _():
      dispatch(l, wait_store)

    @pl.when(l + 3 < L)
    def _():
      def start_next(bb):
        start_gather(l + 3, (bb + 3) % NBUF_B)
      dispatch(l, start_next)

    row_base = b * BT
    tile_base = b * D
    rvecs = [row_base + rb + base16 for rb in range(0, BT, LANES)]
    for d0 in range(0, D, LANES):
      dvecs = [
          tile_base + (d0 + ((base16 + j) & (LANES - 1)))
          for j in range(LANES)
      ]
      for kb, rb in enumerate(range(0, BT, LANES)):
        cvec = rb + base16
        for j in range(LANES):
          diag = d0 + ((base16 + j) & (LANES - 1))
          vec = plsc.load_gather(rows_v, [rvecs[kb], diag])
          plsc.store_scatter(tile_v, [dvecs[j], cvec], vec)

    def start_store(bb):
      pltpu.async_copy(
          tile_v.at[pl.ds(bb * D, D)],
          out_hbm.at[l, :, pl.ds(b0, BT)],
          ssems[bb],
      )
    dispatch(l, start_store)
    return carry

  lax.fori_loop(0, L, body, 0)
  wait_store((L - 4) % NBUF_B)
  wait_store((L - 3) % NBUF_B)
  wait_store((L - 2) % NBUF_B)
  wait_store((L - 1) % NBUF_B)


CHUNK_COLS = 128
NFULL = V // CHUNK_COLS
TAIL = V - NFULL * CHUNK_COLS
CHUNKS_W = NFULL // NW
REM_W = NFULL - CHUNKS_W * NW


@functools.partial(
    pl.kernel,
    out_type=jax.ShapeDtypeStruct((V, DP), jnp.float32),
    mesh=_mesh,
    compiler_params=pltpu.CompilerParams(
        use_tc_tiling_on_sc=True, needs_layout_passes=False
    ),
    scratch_types=[
        pltpu.VMEM((NBUF * D, CHUNK_COLS), jnp.float32),
        pltpu.VMEM((NBUF * CHUNK_COLS, DP), jnp.float32),
        pltpu.VMEM((D, TAIL), jnp.float32),
        [pltpu.SemaphoreType.DMA] * NBUF,
        [pltpu.SemaphoreType.DMA] * NBUF,
    ],
)
def _relayout(tab_t_hbm, tail_t_hbm, out_hbm, src_v, dst_v, tail_v, gsems,
              ssems):
  wid = lax.axis_index("s") * NC + lax.axis_index("c")
  c0 = wid * CHUNKS_W + jnp.minimum(wid, REM_W)
  n = CHUNKS_W + (wid < REM_W).astype(jnp.int32)
  base16 = lax.iota(jnp.int32, LANES)

  def start_read(c, b):
    pltpu.async_copy(
        tab_t_hbm.at[:, pl.ds((c0 + c) * CHUNK_COLS, CHUNK_COLS)],
        src_v.at[pl.ds(b * D, D)],
        gsems[b],
    )

  def wait_read(b):
    pltpu.make_async_copy(
        tab_t_hbm.at[:, pl.ds(0, CHUNK_COLS)],
        src_v.at[pl.ds(0, D)],
        gsems[b],
    ).wait()

  def wait_write(b):
    pltpu.make_async_copy(
        dst_v.at[pl.ds(0, CHUNK_COLS)],
        out_hbm.at[pl.ds(0, CHUNK_COLS)],
        ssems[b],
    ).wait()

  def dispatch(t, fn):
    for b in range(NBUF):
      @pl.when(lax.rem(t, NBUF) == b)
      def _(b=b):
        fn(b)

  @pl.when(n >= 1)
  def _():
    start_read(0, 0)

  @pl.when(n >= 2)
  def _():
    start_read(1, 1)

  def body(c, carry):
    b = lax.rem(c, NBUF)
    dispatch(c, wait_read)

    @pl.when(c >= NBUF)
    def _():
      dispatch(c, wait_write)

    @pl.when(c + 2 < n)
    def _():
      def start_next(bb):
        start_read(c + 2, (bb + 2) % NBUF)
      dispatch(c, start_next)

    src_base = b * D
    dst_base = b * CHUNK_COLS

    def blk_body(t, carry):
      cb = lax.div(t, D // LANES) * LANES
      rb = lax.rem(t, D // LANES) * LANES
      rvec = src_base + rb + base16
      col = rb + base16
      for j in range(LANES):
        cj = cb + ((base16 + j) & (LANES - 1))
        vec = plsc.load_gather(src_v, [rvec, cj])
        plsc.store_scatter(dst_v, [dst_base + cj, col], vec)
      return carry

    lax.fori_loop(0, (CHUNK_COLS // LANES) * (D // LANES), blk_body, 0)

    def start_write(bb):
      pltpu.async_copy(
          dst_v.at[pl.ds(bb * CHUNK_COLS, CHUNK_COLS)],
          out_hbm.at[pl.ds((c0 + c) * CHUNK_COLS, CHUNK_COLS)],
          ssems[bb],
      )
    dispatch(c, start_write)
    return carry

  lax.fori_loop(0, n, body, 0)

  def drain(i, carry):
    dispatch(n - 1 - i, wait_write)
    return carry

  lax.fori_loop(0, jnp.minimum(n, NBUF), drain, 0)

  @pl.when(wid == NW - 1)
  def _():
    pltpu.sync_copy(tail_t_hbm, tail_v)

    def tail_blk(t, carry):
      cb = lax.div(t, D // LANES) * LANES
      rb = lax.rem(t, D // LANES) * LANES
      rvec = rb + base16
      for j in range(LANES):
        cj = cb + ((base16 + j) & (LANES - 1))
        vec = plsc.load_gather(tail_v, [rvec, cj])
        plsc.store_scatter(dst_v, [cj, rvec], vec)
      return carry

    lax.fori_loop(0, (TAIL // LANES) * (D // LANES), tail_blk, 0)
    pltpu.async_copy(
        dst_v.at[pl.ds(0, TAIL)],
        out_hbm.at[pl.ds(NFULL * CHUNK_COLS, TAIL)],
        ssems[0],
    )
    pltpu.make_async_copy(
        dst_v.at[pl.ds(0, TAIL)],
        out_hbm.at[pl.ds(0, TAIL)],
        ssems[0],
    ).wait()


def kernel(indices, table):
  tail_t = table[V - TAIL:].T
  table_pad = _relayout(table.T, tail_t)
  out = _gather(indices.astype(jnp.int32).T, table_pad)
  return jnp.transpose(out, (2, 0, 1))


# speedup vs baseline: 1.0892x; 1.0892x over previous
import functools

import jax
import jax.numpy as jnp
from jax import lax
from jax.experimental import pallas as pl
from jax.experimental.pallas import tpu as pltpu
from jax.experimental.pallas import tpu_sc as plsc

B, L, D = 4096, 200, 64
V = 1000000
DP = 128
NC, NS = 2, 16
NW = NC * NS
ROWS_W = B // NW
SPLITS = ((0, 128), (128, 72))
R = 1
NT = ROWS_W // R
NBUF = 3
LANES = 16

_mesh = plsc.VectorSubcoreMesh(core_axis_name="c", subcore_axis_name="s")


@functools.partial(
    pl.kernel,
    out_type=jax.ShapeDtypeStruct((B, L, DP), jnp.float32),
    mesh=_mesh,
    compiler_params=pltpu.CompilerParams(use_tc_tiling_on_sc=True),
    scratch_types=[
        pltpu.VMEM((ROWS_W, L), jnp.int32),
        pltpu.VMEM((NBUF, R, L, DP), jnp.float32),
        [pltpu.SemaphoreType.DMA] * NBUF,
        [pltpu.SemaphoreType.DMA] * NBUF,
    ],
)
def _gather(idx_hbm, table_hbm, out_hbm, idx_v, rows_v, gsems, ssems):
  wid = lax.axis_index("s") * NC + lax.axis_index("c")
  row0 = wid * ROWS_W
  pltpu.sync_copy(idx_hbm.at[pl.ds(row0, ROWS_W)], idx_v)

  def start_group(t, b):
    for r in range(R):
      for off, size in SPLITS:
        pltpu.async_copy(
            table_hbm.at[idx_v.at[t * R + r, pl.ds(off, size)]],
            rows_v.at[b, r, pl.ds(off, size)],
            gsems[b],
        )

  def wait_gathers(b):
    pltpu.make_async_copy(
        table_hbm.at[pl.ds(0, R * L)], rows_v.at[0], gsems[b]
    ).wait()

  def wait_store(b):
    pltpu.make_async_copy(
        rows_v.at[0], out_hbm.at[pl.ds(0, R)], ssems[b]
    ).wait()

  def dispatch(t, fns):
    for b in range(NBUF):
      @pl.when(lax.rem(t, NBUF) == b)
      def _(b=b):
        fns(b)

  start_group(0, 0)
  start_group(1, 1)

  def body(t, carry):
    def slot(b):
      wait_gathers(b)

      @pl.when(t + 2 < NT)
      def _():
        b2 = (b + 2) % NBUF

        @pl.when(t >= 1)
        def _():
          wait_store(b2)
        start_group(t + 2, b2)

      pltpu.async_copy(
          rows_v.at[b],
          out_hbm.at[pl.ds(row0 + t * R, R)],
          ssems[b],
      )

    dispatch(t, slot)
    return carry

  lax.fori_loop(0, NT, body, 0)
  wait_store((NT - 2) % NBUF)
  wait_store((NT - 1) % NBUF)


CHUNK_COLS = 128
NFULL = V // CHUNK_COLS
TAIL = V - NFULL * CHUNK_COLS
CHUNKS_W = NFULL // NW
REM_W = NFULL - CHUNKS_W * NW


@functools.partial(
    pl.kernel,
    out_type=jax.ShapeDtypeStruct((V, DP), jnp.float32),
    mesh=_mesh,
    compiler_params=pltpu.CompilerParams(
        use_tc_tiling_on_sc=True, needs_layout_passes=False
    ),
    scratch_types=[
        pltpu.VMEM((NBUF * D, CHUNK_COLS), jnp.float32),
        pltpu.VMEM((NBUF * CHUNK_COLS, DP), jnp.float32),
        pltpu.VMEM((D, TAIL), jnp.float32),
        [pltpu.SemaphoreType.DMA] * NBUF,
        [pltpu.SemaphoreType.DMA] * NBUF,
    ],
)
def _relayout(tab_t_hbm, tail_t_hbm, out_hbm, src_v, dst_v, tail_v, gsems,
              ssems):
  wid = lax.axis_index("s") * NC + lax.axis_index("c")
  c0 = wid * CHUNKS_W + jnp.minimum(wid, REM_W)
  n = CHUNKS_W + (wid < REM_W).astype(jnp.int32)
  base16 = lax.iota(jnp.int32, LANES)

  def start_read(c, b):
    pltpu.async_copy(
        tab_t_hbm.at[:, pl.ds((c0 + c) * CHUNK_COLS, CHUNK_COLS)],
        src_v.at[pl.ds(b * D, D)],
        gsems[b],
    )

  def wait_read(b):
    pltpu.make_async_copy(
        tab_t_hbm.at[:, pl.ds(0, CHUNK_COLS)],
        src_v.at[pl.ds(0, D)],
        gsems[b],
    ).wait()

  def wait_write(b):
    pltpu.make_async_copy(
        dst_v.at[pl.ds(0, CHUNK_COLS)],
        out_hbm.at[pl.ds(0, CHUNK_COLS)],
        ssems[b],
    ).wait()

  def dispatch(t, fn):
    for b in range(NBUF):
      @pl.when(lax.rem(t, NBUF) == b)
      def _(b=b):
        fn(b)

  @pl.when(n >= 1)
  def _():
    start_read(0, 0)

  @pl.when(n >= 2)
  def _():
    start_read(1, 1)

  def body(c, carry):
    b = lax.rem(c, NBUF)
    dispatch(c, wait_read)

    @pl.when(c >= NBUF)
    def _():
      dispatch(c, wait_write)

    @pl.when(c + 2 < n)
    def _():
      def start_next(bb):
        start_read(c + 2, (bb + 2) % NBUF)
      dispatch(c, start_next)

    src_base = b * D
    dst_base = b * CHUNK_COLS

    def blk_body(t, carry):
      cb = lax.div(t, D // LANES) * LANES
      rb = lax.rem(t, D // LANES) * LANES
      rvec = src_base + rb + base16
      col = rb + base16
      for j in range(LANES):
        cj = cb + ((base16 + j) & (LANES - 1))
        vec = plsc.load_gather(src_v, [rvec, cj])
        plsc.store_scatter(dst_v, [dst_base + cj, col], vec)
      return carry

    lax.fori_loop(0, (CHUNK_COLS // LANES) * (D // LANES), blk_body, 0)

    def start_write(bb):
      pltpu.async_copy(
          dst_v.at[pl.ds(bb * CHUNK_COLS, CHUNK_COLS)],
          out_hbm.at[pl.ds((c0 + c) * CHUNK_COLS, CHUNK_COLS)],
          ssems[bb],
      )
    dispatch(c, start_write)
    return carry

  lax.fori_loop(0, n, body, 0)

  def drain(i, carry):
    dispatch(n - 1 - i, wait_write)
    return carry

  lax.fori_loop(0, jnp.minimum(n, NBUF), drain, 0)

  @pl.when(wid == NW - 1)
  def _():
    pltpu.sync_copy(tail_t_hbm, tail_v)

    def tail_blk(t, carry):
      cb = lax.div(t, D // LANES) * LANES
      rb = lax.rem(t, D // LANES) * LANES
      rvec = rb + base16
      for j in range(LANES):
        cj = cb + ((base16 + j) & (LANES - 1))
        vec = plsc.load_gather(tail_v, [rvec, cj])
        plsc.store_scatter(dst_v, [cj, rvec], vec)
      return carry

    lax.fori_loop(0, (TAIL // LANES) * (D // LANES), tail_blk, 0)
    pltpu.async_copy(
        dst_v.at[pl.ds(0, TAIL)],
        out_hbm.at[pl.ds(NFULL * CHUNK_COLS, TAIL)],
        ssems[0],
    )
    pltpu.make_async_copy(
        dst_v.at[pl.ds(0, TAIL)],
        out_hbm.at[pl.ds(0, TAIL)],
        ssems[0],
    ).wait()


def kernel(indices, table):
  tail_t = table[V - TAIL:].T
  table_pad = _relayout(table.T, tail_t)
  out = _gather(indices.astype(jnp.int32), table_pad)
  return out[:, :, :D]


# speedup vs baseline: 1.0913x; 1.0019x over previous
import functools

import jax
import jax.numpy as jnp
from jax import lax
from jax.experimental import pallas as pl
from jax.experimental.pallas import tpu as pltpu
from jax.experimental.pallas import tpu_sc as plsc

B, L, D = 4096, 200, 64
V = 1000000
DP = 128
NC, NS = 2, 16
NW = NC * NS
ROWS_W = B // NW
SPLITS = ((0, 128), (128, 72))
R = 1
NT = ROWS_W // R
NBUF = 3
LANES = 16

_mesh = plsc.VectorSubcoreMesh(core_axis_name="c", subcore_axis_name="s")


@functools.partial(
    pl.kernel,
    out_type=jax.ShapeDtypeStruct((B, L, DP), jnp.float32),
    mesh=_mesh,
    compiler_params=pltpu.CompilerParams(use_tc_tiling_on_sc=True),
    scratch_types=[
        pltpu.VMEM((ROWS_W, L), jnp.int32),
        pltpu.VMEM((NBUF, R, L, DP), jnp.float32),
        [pltpu.SemaphoreType.DMA] * NBUF,
        [pltpu.SemaphoreType.DMA] * NBUF,
    ],
)
def _gather(idx_hbm, table_hbm, out_hbm, idx_v, rows_v, gsems, ssems):
  wid = lax.axis_index("s") * NC + lax.axis_index("c")
  row0 = wid * ROWS_W
  pltpu.sync_copy(idx_hbm.at[pl.ds(row0, ROWS_W)], idx_v)

  def start_group(t, b):
    for r in range(R):
      for off, size in SPLITS:
        pltpu.async_copy(
            table_hbm.at[idx_v.at[t * R + r, pl.ds(off, size)]],
            rows_v.at[b, r, pl.ds(off, size)],
            gsems[b],
        )

  def wait_gathers(b):
    pltpu.make_async_copy(
        table_hbm.at[pl.ds(0, R * L)], rows_v.at[0], gsems[b]
    ).wait()

  def wait_store(b):
    pltpu.make_async_copy(
        rows_v.at[0], out_hbm.at[pl.ds(0, R)], ssems[b]
    ).wait()

  def dispatch(t, fns):
    for b in range(NBUF):
      @pl.when(lax.rem(t, NBUF) == b)
      def _(b=b):
        fns(b)

  start_group(0, 0)
  start_group(1, 1)

  def body(t, carry):
    def slot(b):
      wait_gathers(b)

      @pl.when(t + 2 < NT)
      def _():
        b2 = (b + 2) % NBUF

        @pl.when(t >= 1)
        def _():
          wait_store(b2)
        start_group(t + 2, b2)

      pltpu.async_copy(
          rows_v.at[b],
          out_hbm.at[pl.ds(row0 + t * R, R)],
          ssems[b],
      )

    dispatch(t, slot)
    return carry

  lax.fori_loop(0, NT, body, 0)
  wait_store((NT - 2) % NBUF)
  wait_store((NT - 1) % NBUF)


CHUNK_COLS = 128
NFULL = V // CHUNK_COLS
TAIL = V - NFULL * CHUNK_COLS
CHUNKS_W = NFULL // NW
REM_W = NFULL - CHUNKS_W * NW


@functools.partial(
    pl.kernel,
    out_type=jax.ShapeDtypeStruct((V, DP), jnp.float32),
    mesh=_mesh,
    compiler_params=pltpu.CompilerParams(
        use_tc_tiling_on_sc=True, needs_layout_passes=False
    ),
    scratch_types=[
        pltpu.VMEM((NBUF * D, CHUNK_COLS), jnp.float32),
        pltpu.VMEM((NBUF * CHUNK_COLS, DP), jnp.float32),
        pltpu.VMEM((D, TAIL), jnp.float32),
        [pltpu.SemaphoreType.DMA] * NBUF,
        [pltpu.SemaphoreType.DMA] * NBUF,
    ],
)
def _relayout(tab_t_hbm, tail_t_hbm, out_hbm, src_v, dst_v, tail_v, gsems,
              ssems):
  wid = lax.axis_index("s") * NC + lax.axis_index("c")
  c0 = wid * CHUNKS_W + jnp.minimum(wid, REM_W)
  n = CHUNKS_W + (wid < REM_W).astype(jnp.int32)
  base16 = lax.iota(jnp.int32, LANES)

  def start_read(c, b):
    pltpu.async_copy(
        tab_t_hbm.at[:, pl.ds((c0 + c) * CHUNK_COLS, CHUNK_COLS)],
        src_v.at[pl.ds(b * D, D)],
        gsems[b],
    )

  def wait_read(b):
    pltpu.make_async_copy(
        tab_t_hbm.at[:, pl.ds(0, CHUNK_COLS)],
        src_v.at[pl.ds(0, D)],
        gsems[b],
    ).wait()

  def wait_write(b):
    pltpu.make_async_copy(
        dst_v.at[pl.ds(0, CHUNK_COLS)],
        out_hbm.at[pl.ds(0, CHUNK_COLS)],
        ssems[b],
    ).wait()

  def dispatch(t, fn):
    for b in range(NBUF):
      @pl.when(lax.rem(t, NBUF) == b)
      def _(b=b):
        fn(b)

  @pl.when(n >= 1)
  def _():
    start_read(0, 0)

  @pl.when(n >= 2)
  def _():
    start_read(1, 1)

  def body(c, carry):
    b = lax.rem(c, NBUF)
    dispatch(c, wait_read)

    @pl.when(c >= NBUF)
    def _():
      dispatch(c, wait_write)

    @pl.when(c + 2 < n)
    def _():
      def start_next(bb):
        start_read(c + 2, (bb + 2) % NBUF)
      dispatch(c, start_next)

    src_base = b * D
    dst_base = b * CHUNK_COLS

    def blk_body(t, carry):
      cb = lax.div(t, 2) * LANES
      rb = lax.rem(t, 2) * (2 * LANES)
      for rr in (0, LANES):
        rvec = src_base + rb + rr + base16
        col = rb + rr + base16
        for j in range(LANES):
          cj = cb + ((base16 + j) & (LANES - 1))
          vec = plsc.load_gather(src_v, [rvec, cj])
          plsc.store_scatter(dst_v, [dst_base + cj, col], vec)
      return carry

    lax.fori_loop(0, (CHUNK_COLS // LANES) * 2, blk_body, 0)

    def start_write(bb):
      pltpu.async_copy(
          dst_v.at[pl.ds(bb * CHUNK_COLS, CHUNK_COLS)],
          out_hbm.at[pl.ds((c0 + c) * CHUNK_COLS, CHUNK_COLS)],
          ssems[bb],
      )
    dispatch(c, start_write)
    return carry

  lax.fori_loop(0, n, body, 0)

  def drain(i, carry):
    dispatch(n - 1 - i, wait_write)
    return carry

  lax.fori_loop(0, jnp.minimum(n, NBUF), drain, 0)

  @pl.when(wid == NW - 1)
  def _():
    pltpu.sync_copy(tail_t_hbm, tail_v)

    def tail_blk(t, carry):
      cb = lax.div(t, D // LANES) * LANES
      rb = lax.rem(t, D // LANES) * LANES
      rvec = rb + base16
      for j in range(LANES):
        cj = cb + ((base16 + j) & (LANES - 1))
        vec = plsc.load_gather(tail_v, [rvec, cj])
        plsc.store_scatter(dst_v, [cj, rvec], vec)
      return carry

    lax.fori_loop(0, (TAIL // LANES) * (D // LANES), tail_blk, 0)
    pltpu.async_copy(
        dst_v.at[pl.ds(0, TAIL)],
        out_hbm.at[pl.ds(NFULL * CHUNK_COLS, TAIL)],
        ssems[0],
    )
    pltpu.make_async_copy(
        dst_v.at[pl.ds(0, TAIL)],
        out_hbm.at[pl.ds(0, TAIL)],
        ssems[0],
    ).wait()


def kernel(indices, table):
  tail_t = table[V - TAIL:].T
  table_pad = _relayout(table.T, tail_t)
  out = _gather(indices.astype(jnp.int32), table_pad)
  return out[:, :, :D]
